# Initial kernel scaffold; baseline (speedup 1.0000x reference)
import jax
import jax.numpy as jnp
from jax.experimental import pallas as pl


def kernel(pos, edge_index):
    raise NotImplementedError("write your pallas kernel here")


# speedup vs baseline: 101.4826x; 101.4826x over previous
import functools

import jax
import jax.numpy as jnp
from jax import lax
from jax.experimental import pallas as pl
from jax.experimental.pallas import tpu as pltpu
from jax.experimental.pallas import tpu_sc as plsc

N_LANES = 16
ROW_W = 128
ROWS_PER_CHUNK = 16
CHUNK = ROW_W * ROWS_PER_CHUNK
N_WORKERS = 32


@functools.partial(jax.jit, static_argnames=("n_chunks",))
def _sc_distance(px, py, pz, src2d, dst2d, n_chunks):
    n_rows = src2d.shape[0]
    n_nodes = px.shape[0]
    rows_per_tile = n_rows // N_WORKERS

    mesh = plsc.VectorSubcoreMesh(core_axis_name="c", subcore_axis_name="s")

    @functools.partial(
        pl.kernel,
        out_type=jax.ShapeDtypeStruct((n_rows, ROW_W), jnp.float32),
        mesh=mesh,
        scratch_types=[
            pltpu.VMEM((ROWS_PER_CHUNK, ROW_W), jnp.int32),
            pltpu.VMEM((ROWS_PER_CHUNK, ROW_W), jnp.int32),
            [pltpu.VMEM((ROWS_PER_CHUNK, ROW_W), jnp.float32)
             for _ in range(6)],
            pltpu.VMEM((ROWS_PER_CHUNK, ROW_W), jnp.float32),
            [pltpu.VMEM_SHARED((n_nodes,), jnp.float32)
             for _ in range(3)],
            pltpu.SemaphoreType.DMA,
            pltpu.SemaphoreType.DMA,
        ],
    )
    def k(px_hbm, py_hbm, pz_hbm, src_hbm, dst_hbm, out_hbm,
          idx_s, idx_d, comps, w_v, tabs, sem_tab, sem):
        cid = lax.axis_index("c")
        sid = lax.axis_index("s")
        wid = sid * 2 + cid

        @pl.when(sid == 0)
        def _():
            for c, src_ref in enumerate((px_hbm, py_hbm, pz_hbm)):
                pltpu.async_copy(src_ref, tabs[c], sem_tab).wait()
        plsc.subcore_barrier()

        base_row = wid * rows_per_tile

        def chunk_body(kc, _):
            row0 = base_row + kc * ROWS_PER_CHUNK
            pltpu.sync_copy(src_hbm.at[pl.ds(row0, ROWS_PER_CHUNK)], idx_s)
            pltpu.sync_copy(dst_hbm.at[pl.ds(row0, ROWS_PER_CHUNK)], idx_d)
            cps = []
            for j in range(ROWS_PER_CHUNK):
                for c in range(3):
                    cps.append(pltpu.async_copy(
                        tabs[c].at[idx_s.at[j]], comps[c].at[j], sem))
                    cps.append(pltpu.async_copy(
                        tabs[c].at[idx_d.at[j]], comps[3 + c].at[j], sem))
            for cp in cps:
                cp.wait()

            def grp_body(g, _):
                j = g // 8
                c0 = (g % 8) * N_LANES
                dx = comps[0][j, pl.ds(c0, N_LANES)] - comps[3][j, pl.ds(c0, N_LANES)]
                dy = comps[1][j, pl.ds(c0, N_LANES)] - comps[4][j, pl.ds(c0, N_LANES)]
                dz = comps[2][j, pl.ds(c0, N_LANES)] - comps[5][j, pl.ds(c0, N_LANES)]
                ssq = dx * dx + dy * dy + dz * dz
                x = jnp.maximum(ssq, jnp.float32(1e-36))
                xi = lax.bitcast_convert_type(x, jnp.int32)
                seed = jnp.full((N_LANES,), 0x5F3759DF, jnp.int32) - (xi >> 1)
                g0 = lax.bitcast_convert_type(seed, jnp.float32)
                h = x * jnp.float32(0.5)
                g1 = g0 * (jnp.float32(1.5) - h * g0 * g0)
                g2 = g1 * (jnp.float32(1.5) - h * g1 * g1)
                g3 = g2 * (jnp.float32(1.5) - h * g2 * g2)
                w_v[j, pl.ds(c0, N_LANES)] = x * g3
                return 0

            lax.fori_loop(0, CHUNK // N_LANES, grp_body, 0)
            pltpu.sync_copy(w_v, out_hbm.at[pl.ds(row0, ROWS_PER_CHUNK)])
            return 0

        lax.fori_loop(0, n_chunks, chunk_body, 0)

    return k(px, py, pz, src2d, dst2d)


def kernel(pos, edge_index):
    src = edge_index[0]
    dst = edge_index[1]
    e = src.shape[0]
    per_round = N_WORKERS * CHUNK
    n_chunks = -(-e // per_round)
    e_pad = n_chunks * per_round
    pad = e_pad - e
    if pad:
        zpad = jnp.zeros((pad,), src.dtype)
        src_p = jnp.concatenate([src, zpad])
        dst_p = jnp.concatenate([dst, zpad])
    else:
        src_p, dst_p = src, dst
    w2d = _sc_distance(pos[:, 0], pos[:, 1], pos[:, 2],
                       src_p.reshape(-1, ROW_W), dst_p.reshape(-1, ROW_W),
                       n_chunks)
    w = w2d.reshape(-1)
    if pad:
        w = w[:e]
    return (edge_index, w)


# speedup vs baseline: 117.9771x; 1.1625x over previous
import functools

import jax
import jax.numpy as jnp
from jax import lax
from jax.experimental import pallas as pl
from jax.experimental.pallas import tpu as pltpu
from jax.experimental.pallas import tpu_sc as plsc

N_LANES = 16
ROW_W = 128
ROWS_PER_CHUNK = 16
CHUNK = ROW_W * ROWS_PER_CHUNK
N_WORKERS = 32


@jax.jit
def _sc_distance(px, py, pz, ei3):
    n_rows = ei3.shape[1]
    n_nodes = px.shape[0]
    n_chunks = n_rows // ROWS_PER_CHUNK
    base_chunks = n_chunks // N_WORKERS
    n_extra = n_chunks - base_chunks * N_WORKERS

    mesh = plsc.VectorSubcoreMesh(core_axis_name="c", subcore_axis_name="s")

    @functools.partial(
        pl.kernel,
        out_type=jax.ShapeDtypeStruct((n_rows, ROW_W), jnp.float32),
        mesh=mesh,
        scratch_types=[
            pltpu.VMEM((ROWS_PER_CHUNK, ROW_W), jnp.int32),
            pltpu.VMEM((ROWS_PER_CHUNK, ROW_W), jnp.int32),
            [pltpu.VMEM((ROWS_PER_CHUNK, ROW_W), jnp.float32)
             for _ in range(6)],
            pltpu.VMEM((ROWS_PER_CHUNK, ROW_W), jnp.float32),
            [pltpu.VMEM_SHARED((n_nodes,), jnp.float32)
             for _ in range(3)],
            pltpu.SemaphoreType.DMA,
            pltpu.SemaphoreType.DMA,
        ],
    )
    def k(px_hbm, py_hbm, pz_hbm, ei_hbm, out_hbm,
          idx_s, idx_d, comps, w_v, tabs, sem_tab, sem):
        cid = lax.axis_index("c")
        sid = lax.axis_index("s")
        wid = sid * 2 + cid

        @pl.when(sid == 0)
        def _():
            for c, src_ref in enumerate((px_hbm, py_hbm, pz_hbm)):
                pltpu.async_copy(src_ref, tabs[c], sem_tab).wait()
        plsc.subcore_barrier()

        n_mine = base_chunks + jnp.where(wid < n_extra, 1, 0)

        def chunk_body(kc, _):
            row0 = (kc * N_WORKERS + wid) * ROWS_PER_CHUNK
            pltpu.sync_copy(ei_hbm.at[0, pl.ds(row0, ROWS_PER_CHUNK)], idx_s)
            pltpu.sync_copy(ei_hbm.at[1, pl.ds(row0, ROWS_PER_CHUNK)], idx_d)
            cps = []
            for j in range(ROWS_PER_CHUNK):
                for c in range(3):
                    cps.append(pltpu.async_copy(
                        tabs[c].at[idx_s.at[j]], comps[c].at[j], sem))
                    cps.append(pltpu.async_copy(
                        tabs[c].at[idx_d.at[j]], comps[3 + c].at[j], sem))
            for cp in cps:
                cp.wait()

            def grp_body(g, _):
                j = g // 8
                c0 = (g % 8) * N_LANES
                dx = comps[0][j, pl.ds(c0, N_LANES)] - comps[3][j, pl.ds(c0, N_LANES)]
                dy = comps[1][j, pl.ds(c0, N_LANES)] - comps[4][j, pl.ds(c0, N_LANES)]
                dz = comps[2][j, pl.ds(c0, N_LANES)] - comps[5][j, pl.ds(c0, N_LANES)]
                ssq = dx * dx + dy * dy + dz * dz
                x = jnp.maximum(ssq, jnp.float32(1e-36))
                xi = lax.bitcast_convert_type(x, jnp.int32)
                seed = jnp.full((N_LANES,), 0x5F3759DF, jnp.int32) - (xi >> 1)
                g0 = lax.bitcast_convert_type(seed, jnp.float32)
                h = x * jnp.float32(0.5)
                g1 = g0 * (jnp.float32(1.5) - h * g0 * g0)
                g2 = g1 * (jnp.float32(1.5) - h * g1 * g1)
                g3 = g2 * (jnp.float32(1.5) - h * g2 * g2)
                w_v[j, pl.ds(c0, N_LANES)] = x * g3
                return 0

            lax.fori_loop(0, CHUNK // N_LANES, grp_body, 0)
            pltpu.sync_copy(w_v, out_hbm.at[pl.ds(row0, ROWS_PER_CHUNK)])
            return 0

        lax.fori_loop(0, n_mine, chunk_body, 0)

    return k(px, py, pz, ei3)


def kernel(pos, edge_index):
    e = edge_index.shape[1]
    assert e % CHUNK == 0, "edge count must be a multiple of 2048"
    ei3 = edge_index.reshape(2, e // ROW_W, ROW_W)
    w2d = _sc_distance(pos[:, 0], pos[:, 1], pos[:, 2], ei3)
    return (edge_index, w2d.reshape(-1))


# speedup vs baseline: 143.6644x; 1.2177x over previous
import functools

import jax
import jax.numpy as jnp
from jax import lax
from jax.experimental import pallas as pl
from jax.experimental.pallas import tpu as pltpu
from jax.experimental.pallas import tpu_sc as plsc

N_LANES = 16
ROW_W = 128
ROWS_PER_CHUNK = 16
CHUNK = ROW_W * ROWS_PER_CHUNK
N_WORKERS = 32


@jax.jit
def _sc_distance(px, py, pz, ei3):
    n_rows = ei3.shape[1]
    n_nodes = px.shape[0]
    n_chunks = n_rows // ROWS_PER_CHUNK
    cpt = -(-n_chunks // N_WORKERS)
    assert cpt % 2 == 0 and cpt >= 4

    mesh = plsc.VectorSubcoreMesh(core_axis_name="c", subcore_axis_name="s")

    @functools.partial(
        pl.kernel,
        out_type=jax.ShapeDtypeStruct((n_rows, ROW_W), jnp.float32),
        mesh=mesh,
        scratch_types=[
            [pltpu.VMEM((ROWS_PER_CHUNK, ROW_W), jnp.int32)
             for _ in range(4)],
            [pltpu.VMEM((ROWS_PER_CHUNK, ROW_W), jnp.float32)
             for _ in range(12)],
            [pltpu.VMEM((ROWS_PER_CHUNK, ROW_W), jnp.float32)
             for _ in range(2)],
            [pltpu.VMEM_SHARED((n_nodes,), jnp.float32)
             for _ in range(3)],
            pltpu.SemaphoreType.DMA,
            [pltpu.SemaphoreType.DMA for _ in range(2)],
            [pltpu.SemaphoreType.DMA for _ in range(2)],
        ],
    )
    def k(px_hbm, py_hbm, pz_hbm, ei_hbm, out_hbm,
          idxs, comps, wvs, tabs, sem_tab, sem_g, sem_w):
        cid = lax.axis_index("c")
        sid = lax.axis_index("s")
        wid = sid * 2 + cid

        @pl.when(sid == 0)
        def _():
            for c, src_ref in enumerate((px_hbm, py_hbm, pz_hbm)):
                pltpu.async_copy(src_ref, tabs[c], sem_tab).wait()
        plsc.subcore_barrier()

        def row0_of(kc):
            c_id = jnp.minimum(kc * N_WORKERS + wid, n_chunks - 1)
            return c_id * ROWS_PER_CHUNK

        def fetch(kc, b):
            row0 = row0_of(kc)
            pltpu.sync_copy(ei_hbm.at[0, pl.ds(row0, ROWS_PER_CHUNK)],
                            idxs[2 * b])
            pltpu.sync_copy(ei_hbm.at[1, pl.ds(row0, ROWS_PER_CHUNK)],
                            idxs[2 * b + 1])
            for j in range(ROWS_PER_CHUNK):
                for c in range(3):
                    pltpu.async_copy(tabs[c].at[idxs[2 * b].at[j]],
                                     comps[6 * b + c].at[j], sem_g[b])
                    pltpu.async_copy(tabs[c].at[idxs[2 * b + 1].at[j]],
                                     comps[6 * b + 3 + c].at[j], sem_g[b])

        def drain_gathers(b):
            for j in range(ROWS_PER_CHUNK):
                for c in range(6):
                    pltpu.make_async_copy(tabs[0].at[idxs[2 * b].at[j]],
                                          comps[6 * b + c].at[j],
                                          sem_g[b]).wait()

        def wait_store(b, kc):
            pltpu.make_async_copy(wvs[b], out_hbm.at[pl.ds(row0_of(kc),
                                                           ROWS_PER_CHUNK)],
                                  sem_w[b]).wait()

        def compute_store(kc, b):
            def grp_body(g, _):
                j = g // 8
                c0 = (g % 8) * N_LANES
                cb = comps[6 * b:6 * b + 6]
                dx = cb[0][j, pl.ds(c0, N_LANES)] - cb[3][j, pl.ds(c0, N_LANES)]
                dy = cb[1][j, pl.ds(c0, N_LANES)] - cb[4][j, pl.ds(c0, N_LANES)]
                dz = cb[2][j, pl.ds(c0, N_LANES)] - cb[5][j, pl.ds(c0, N_LANES)]
                ssq = dx * dx + dy * dy + dz * dz
                x = jnp.maximum(ssq, jnp.float32(1e-36))
                xi = lax.bitcast_convert_type(x, jnp.int32)
                seed = jnp.full((N_LANES,), 0x5F3759DF, jnp.int32) - (xi >> 1)
                g0 = lax.bitcast_convert_type(seed, jnp.float32)
                h = x * jnp.float32(0.5)
                g1 = g0 * (jnp.float32(1.5) - h * g0 * g0)
                g2 = g1 * (jnp.float32(1.5) - h * g1 * g1)
                g3 = g2 * (jnp.float32(1.5) - h * g2 * g2)
                wvs[b][j, pl.ds(c0, N_LANES)] = x * g3
                return 0

            lax.fori_loop(0, CHUNK // N_LANES, grp_body, 0)
            pltpu.async_copy(wvs[b],
                             out_hbm.at[pl.ds(row0_of(kc), ROWS_PER_CHUNK)],
                             sem_w[b])

        fetch(0, 0)
        fetch(1, 1)
        drain_gathers(0)
        compute_store(0, 0)
        fetch(2, 0)
        drain_gathers(1)
        compute_store(1, 1)
        fetch(3, 1)

        def pair_body(kp, _):
            for b in (0, 1):
                kc = kp * 2 + b
                drain_gathers(b)
                wait_store(b, kc)
                compute_store(kc, b)
                nxt = kc + 2

                @pl.when(nxt < cpt)
                def _():
                    fetch(nxt, b)
            return 0

        lax.fori_loop(1, cpt // 2, pair_body, 0)

        wait_store(0, cpt - 2)
        wait_store(1, cpt - 1)

    return k(px, py, pz, ei3)


def kernel(pos, edge_index):
    e = edge_index.shape[1]
    assert e % CHUNK == 0, "edge count must be a multiple of 2048"
    ei3 = edge_index.reshape(2, e // ROW_W, ROW_W)
    w2d = _sc_distance(pos[:, 0], pos[:, 1], pos[:, 2], ei3)
    return (edge_index, w2d.reshape(-1))
